# baseline (speedup 1.0000x reference)
import jax
import jax.numpy as jnp
from jax import lax
from jax.experimental import pallas as pl
from jax.experimental.pallas import tpu as pltpu
from jax.experimental.pallas import tpu_sc as plsc

VOCAB = 1000000
DIM = 64
SEQ = 200
BATCH = 4096

NC, NS = 2, 16
NW = NC * NS
BBLK = BATCH // NW
SC_CHUNK = 4
N_CHUNK = SEQ // SC_CHUNK


def _body(tok_hbm, bias_hbm, table_hbm, out_hbm, tok_v, bias_v, rows_v,
          gsem0, gsem1, osem0, osem1):
    wid = lax.axis_index("s") * NC + lax.axis_index("c")
    pltpu.sync_copy(tok_hbm.at[wid], tok_v)
    pltpu.sync_copy(bias_hbm.at[pl.ds(wid * BBLK, BBLK)], bias_v)
    gsems = (gsem0, gsem1)
    osems = (osem0, osem1)

    def issue_gathers(c, b):
        s0 = c * SC_CHUNK
        for i in range(SC_CHUNK):
            pltpu.async_copy(table_hbm.at[tok_v.at[s0 + i]],
                             rows_v.at[b, i], gsems[b])

    def wait_gathers(c, b):
        for i in range(SC_CHUNK):
            pltpu.make_async_copy(table_hbm.at[tok_v.at[c * SC_CHUNK + i]],
                                  rows_v.at[b, i], gsems[b]).wait()

    def issue_writes(c, b):
        s0 = c * SC_CHUNK
        pltpu.async_copy(
            rows_v.at[b],
            out_hbm.at[pl.ds(s0, SC_CHUNK), pl.ds(wid * BBLK, BBLK)],
            osems[b])

    def wait_writes(c, b):
        s0 = c * SC_CHUNK
        pltpu.make_async_copy(
            rows_v.at[b],
            out_hbm.at[pl.ds(s0, SC_CHUNK), pl.ds(wid * BBLK, BBLK)],
            osems[b]).wait()

    def add_bias(b):
        def jloop(j, _):
            for k in range(DIM // 16):
                bv = bias_v[j, pl.ds(k * 16, 16)]
                for i in range(SC_CHUNK):
                    rows_v[b, i, j, pl.ds(k * 16, 16)] = (
                        rows_v[b, i, j, pl.ds(k * 16, 16)] + bv)
            return 0

        lax.fori_loop(0, BBLK, jloop, 0)

    issue_gathers(0, 0)

    def outer(cc, _):
        for b in range(2):
            c = cc * 2 + b

            @pl.when(c >= 1)
            def _():
                wait_writes(c - 1, 1 - b)

            @pl.when(c + 1 < N_CHUNK)
            def _():
                issue_gathers(c + 1, 1 - b)

            wait_gathers(c, b)
            add_bias(b)
            issue_writes(c, b)
        return 0

    lax.fori_loop(0, N_CHUNK // 2, outer, 0)
    wait_writes(N_CHUNK - 1, 1)


@jax.jit
def _run(tok_blocked, bias, table):
    mesh = plsc.VectorSubcoreMesh(core_axis_name="c", subcore_axis_name="s")
    f = pl.kernel(
        _body,
        out_type=jax.ShapeDtypeStruct((SEQ, BATCH, DIM), jnp.float32),
        mesh=mesh,
        scratch_types=[
            pltpu.VMEM((SEQ, BBLK), jnp.int32),
            pltpu.VMEM((BBLK, DIM), jnp.float32),
            pltpu.VMEM((2, SC_CHUNK, BBLK, DIM), jnp.float32),
            pltpu.SemaphoreType.DMA,
            pltpu.SemaphoreType.DMA,
            pltpu.SemaphoreType.DMA,
            pltpu.SemaphoreType.DMA,
        ],
        compiler_params=pltpu.CompilerParams(use_tc_tiling_on_sc=False,
                                             skip_device_barrier=True),
    )
    return f(tok_blocked, bias, table)


def kernel(tokens, table, condition_bias):
    tok_blocked = (tokens.astype(jnp.int32)
                   .reshape(SEQ, NW, BBLK)
                   .transpose(1, 0, 2))
    return _run(tok_blocked, condition_bias, table)


# speedup vs baseline: 1.0830x; 1.0830x over previous
import jax
import jax.numpy as jnp
from jax import lax
from jax.experimental import pallas as pl
from jax.experimental.pallas import tpu as pltpu
from jax.experimental.pallas import tpu_sc as plsc

VOCAB = 1000000
DIM = 64
SEQ = 200
BATCH = 4096

NC, NS = 2, 16
NW = NC * NS
BBLK = BATCH // NW


def _body(tok_hbm, bias_hbm, table_hbm, out_hbm, tok_v, bias_v, gbuf, obuf,
          gsem0, gsem1, osem0, osem1):
    wid = lax.axis_index("s") * NC + lax.axis_index("c")
    pltpu.sync_copy(tok_hbm.at[wid], tok_v)
    pltpu.sync_copy(bias_hbm.at[pl.ds(wid * BBLK, BBLK)], bias_v)
    gsems = (gsem0, gsem1)
    osems = (osem0, osem1)

    def issue_gather(s, b):
        pltpu.async_copy(table_hbm.at[tok_v.at[s]], gbuf.at[b], gsems[b])

    def wait_gather(s, b):
        pltpu.make_async_copy(table_hbm.at[tok_v.at[s]], gbuf.at[b],
                              gsems[b]).wait()

    def issue_write(s, b):
        pltpu.async_copy(obuf.at[b],
                         out_hbm.at[s, pl.ds(wid * BBLK, BBLK)], osems[b])

    def wait_write(s, b):
        pltpu.make_async_copy(obuf.at[b],
                              out_hbm.at[s, pl.ds(wid * BBLK, BBLK)],
                              osems[b]).wait()

    def add_bias(b):
        def jloop(j, _):
            for k in range(DIM // 16):
                obuf[b, j, pl.ds(k * 16, 16)] = (
                    gbuf[b, j, pl.ds(k * 16, 16)]
                    + bias_v[j, pl.ds(k * 16, 16)])
            return 0

        lax.fori_loop(0, BBLK, jloop, 0)

    issue_gather(0, 0)

    def outer(cc, _):
        for b in range(2):
            s = cc * 2 + b

            @pl.when(s >= 2)
            def _():
                wait_write(s - 2, b)

            @pl.when(s + 1 < SEQ)
            def _():
                issue_gather(s + 1, 1 - b)

            wait_gather(s, b)
            add_bias(b)
            issue_write(s, b)
        return 0

    lax.fori_loop(0, SEQ // 2, outer, 0)
    wait_write(SEQ - 2, 0)
    wait_write(SEQ - 1, 1)


@jax.jit
def _run(tok_blocked, bias, table_padded):
    mesh = plsc.VectorSubcoreMesh(core_axis_name="c", subcore_axis_name="s")
    f = pl.kernel(
        _body,
        out_type=jax.ShapeDtypeStruct((SEQ, BATCH, DIM), jnp.float32),
        mesh=mesh,
        scratch_types=[
            pltpu.VMEM((SEQ, BBLK), jnp.int32),
            pltpu.VMEM((BBLK, DIM), jnp.float32),
            pltpu.VMEM((2, BBLK, 128), jnp.float32),
            pltpu.VMEM((2, BBLK, DIM), jnp.float32),
            pltpu.SemaphoreType.DMA,
            pltpu.SemaphoreType.DMA,
            pltpu.SemaphoreType.DMA,
            pltpu.SemaphoreType.DMA,
        ],
        compiler_params=pltpu.CompilerParams(use_tc_tiling_on_sc=True),
    )
    return f(tok_blocked, bias, table_padded)


def kernel(tokens, table, condition_bias):
    tok_blocked = (tokens.astype(jnp.int32)
                   .reshape(SEQ, NW, BBLK)
                   .transpose(1, 0, 2))
    table_padded = jnp.pad(table, ((0, 0), (0, 128 - DIM)))
    return _run(tok_blocked, condition_bias, table_padded)
